# baseline (speedup 1.0000x reference)
import functools

import jax
import jax.numpy as jnp
from jax import lax
from jax.experimental import pallas as pl
from jax.experimental.pallas import tpu as pltpu
from jax.experimental.pallas import tpu_sc as plsc

N_NODES = 50000
N_EDGES = 800000
FEAT_DIM = 128
LATENT_DIM = 64
HIDDEN = 128
NUM_CLASS = 2
MAX_LV = 3
N_GRAPHS = 128

NPAD = 50176
BN = 512
GRID_N = NPAD // BN

ECHUNK = 128
NT = 16
NB = 3
ROWS_PER_TILE = 393
EROWS = ROWS_PER_TILE * NT
BLK = NB * ECHUNK
G_OUT = ROWS_PER_TILE // NB
EPAD = EROWS * ECHUNK

HALF = N_NODES // 2
ACC_ROWS = 25088
TRASH = 25080
DST_PAD = 1 << 20
ZSTRIPE = ACC_ROWS // NT
COPY_STRIPE = 1560


REG = 51072
PART_TOT = 2 * NT * REG
PBUF = 800


def _sc_partition_body(src_hbm, dst_hbm, srcp_hbm, dstp_hbm, cnt_hbm,
                       srcb0, srcb1, srcb2, dstb0, dstb1, dstb2,
                       pbuf, qbuf, cntv, srcp_s, dstp_s,
                       isemA, isemB):
    srcbs = [srcb0, srcb1, srcb2]
    dstbs = [dstb0, dstb1, dstb2]
    c = lax.axis_index("c")
    t = lax.axis_index("s")
    cbase = c * HALF
    tebase = t * (ROWS_PER_TILE * ECHUNK)

    def outer(g, carry):
        cnt, outpos = carry
        idx_hs = []
        for b in range(NB):
            base = pl.multiple_of(tebase + g * BLK + b * ECHUNK, 8)
            idx_hs.append(pltpu.async_copy(
                src_hbm.at[pl.ds(base, ECHUNK)], srcbs[b], isemA))
            idx_hs.append(pltpu.async_copy(
                dst_hbm.at[pl.ds(base, ECHUNK)], dstbs[b], isemB))
        for h in idx_hs:
            h.wait()
        ii = lax.iota(jnp.int32, 16)
        for b in range(NB):
            for i in range(8):
                s = srcbs[b][pl.ds(i * 16, 16)]
                d = dstbs[b][pl.ds(i * 16, 16)]
                rel = d - cbase
                ok = jnp.logical_and(rel >= 0, rel < HALF)
                _, sv, _ = plsc.sort_key_val(ii, s, mask=ok)
                _, rv, _ = plsc.sort_key_val(ii, rel, mask=ok)
                pbuf[pl.ds(cnt, 16)] = sv
                qbuf[pl.ds(cnt, 16)] = rv
                cnt = cnt + jnp.max(plsc.all_reduce_population_count(ok))
        do = cnt >= 384

        @pl.when(do)
        def _():
            pltpu.sync_copy(pbuf.at[pl.ds(0, 384)],
                            srcp_s.at[t, pl.ds(pl.multiple_of(outpos, 8), 384)])
            pltpu.sync_copy(qbuf.at[pl.ds(0, 384)],
                            dstp_s.at[t, pl.ds(pl.multiple_of(outpos, 8), 384)])
            for i in range(24):
                pbuf[pl.ds(i * 16, 16)] = pbuf[pl.ds(384 + i * 16, 16)]
                qbuf[pl.ds(i * 16, 16)] = qbuf[pl.ds(384 + i * 16, 16)]

        cnt = jnp.where(do, cnt - 384, cnt)
        outpos = jnp.where(do, outpos + 384, outpos)
        return (cnt, outpos)

    cnt, outpos = lax.fori_loop(
        0, G_OUT, outer, (jnp.int32(0), jnp.int32(0)))

    zero16 = jnp.zeros((16,), jnp.int32)
    trash16 = jnp.full((16,), TRASH, jnp.int32)
    for i in range(24):
        pbuf[pl.ds(cnt + i * 16, 16)] = zero16
        qbuf[pl.ds(cnt + i * 16, 16)] = trash16
    pltpu.sync_copy(pbuf.at[pl.ds(0, 384)], srcp_s.at[t, pl.ds(pl.multiple_of(outpos, 8), 384)])
    pltpu.sync_copy(qbuf.at[pl.ds(0, 384)], dstp_s.at[t, pl.ds(pl.multiple_of(outpos, 8), 384)])
    outpos = outpos + 384

    cntv[pl.ds(0, 16)] = jnp.full((16,), outpos, jnp.int32)
    pltpu.sync_copy(cntv, cnt_hbm.at[c * NT + t])
    wreg = pl.multiple_of((c * NT + t) * REG, 8)
    pltpu.sync_copy(srcp_s.at[t], srcp_hbm.at[pl.ds(wreg, REG)])
    pltpu.sync_copy(dstp_s.at[t], dstp_hbm.at[pl.ds(wreg, REG)])


@functools.cache
def _get_sc_partition():
    return functools.partial(
        pl.kernel,
        mesh=plsc.VectorSubcoreMesh(core_axis_name="c", subcore_axis_name="s"),
        out_type=[
            jax.ShapeDtypeStruct((PART_TOT,), jnp.int32),
            jax.ShapeDtypeStruct((PART_TOT,), jnp.int32),
            jax.ShapeDtypeStruct((2 * NT, 16), jnp.int32),
        ],
        scratch_types=(
            [pltpu.VMEM((ECHUNK,), jnp.int32)] * (2 * NB)
            + [pltpu.VMEM((PBUF,), jnp.int32)] * 2
            + [pltpu.VMEM((16,), jnp.int32)]
            + [pltpu.VMEM_SHARED((NT, REG), jnp.int32)] * 2
            + [pltpu.SemaphoreType.DMA] * 2
        ),
        compiler_params=pltpu.CompilerParams(use_tc_tiling_on_sc=False,
                                             needs_layout_passes=False),
    )(_sc_partition_body)


def _sc_segment_sum_body(cur_hbm, srcp_hbm, dstp_hbm, cnt_hbm, pool_hbm,
                         srcb0, srcb1, srcb2,
                         dstbuf0, dstbuf1, dstbuf2,
                         rows0, rows1, rows2,
                         cntbuf, acc,
                         isemA, isemB,
                         gsem0, gsem1, gsem2,
                         ssem0, ssem1, ssem2):
    srcbs = [srcb0, srcb1, srcb2]
    rows = [rows0, rows1, rows2]
    dstbufs = [dstbuf0, dstbuf1, dstbuf2]
    gsems = [gsem0, gsem1, gsem2]
    ssems = [ssem0, ssem1, ssem2]
    c = lax.axis_index("c")
    t = lax.axis_index("s")
    cbase = c * HALF
    rbase = (c * NT + t) * REG

    pltpu.sync_copy(cnt_hbm.at[c * NT + t], cntbuf)
    nsteps = cntbuf[pl.ds(0, 16)][0] // BLK

    zero16 = jnp.zeros((16,), jnp.float32)

    def zfill(i, carry):
        rows0[i // 4, pl.ds((i % 4) * 16, 16)] = zero16
        return carry

    lax.fori_loop(0, 512, zfill, 0)

    for k in range(12):
        pltpu.sync_copy(
            rows0,
            acc.at[pl.ds(pl.multiple_of(t * ZSTRIPE + k * 128, 8), 128)])
    pltpu.sync_copy(
        rows0.at[pl.ds(0, 32)],
        acc.at[pl.ds(pl.multiple_of(t * ZSTRIPE + 12 * 128, 8), 32)])

    @pl.when(jnp.logical_and(c == 1, t == NT - 1))
    def _():
        pltpu.sync_copy(rows0, pool_hbm.at[pl.ds(N_NODES, 128)])
        pltpu.sync_copy(rows0.at[pl.ds(0, 48)],
                        pool_hbm.at[pl.ds(N_NODES + 128, 48)])

    plsc.subcore_barrier()

    def outer(g, carry):
        idx_hs = []
        for b in range(NB):
            base = pl.multiple_of(rbase + g * BLK + b * ECHUNK, 8)
            idx_hs.append(pltpu.async_copy(
                srcp_hbm.at[pl.ds(base, ECHUNK)], srcbs[b], isemA))
            idx_hs.append(pltpu.async_copy(
                dstp_hbm.at[pl.ds(base, ECHUNK)], dstbufs[b].at[0], isemB))
        for b in range(NB):
            idx_hs[2 * b].wait()
            idx_hs[2 * b + 1].wait()

            @pl.when(g > 0)
            def _():
                pltpu.make_async_copy(
                    rows[b], acc.at[dstbufs[b].at[0]], ssems[b]).wait()

            pltpu.async_copy(cur_hbm.at[srcbs[b]], rows[b], gsems[b])
        for b in range(NB):
            pltpu.make_async_copy(
                cur_hbm.at[srcbs[b]], rows[b], gsems[b]).wait()
            pltpu.async_copy(rows[b], acc.at[dstbufs[b].at[0]],
                             ssems[b], add=True)
        return carry

    lax.fori_loop(0, nsteps, outer, 0)
    for b in range(NB):
        pltpu.make_async_copy(rows[b], acc.at[dstbufs[b].at[0]],
                              ssems[b]).wait()

    plsc.subcore_barrier()

    pltpu.sync_copy(
        acc.at[pl.ds(pl.multiple_of(t * COPY_STRIPE, 8), COPY_STRIPE)],
        pool_hbm.at[pl.ds(pl.multiple_of(cbase + t * COPY_STRIPE, 8),
                          COPY_STRIPE)])

    @pl.when(t == NT - 1)
    def _():
        pltpu.sync_copy(
            acc.at[pl.ds(NT * COPY_STRIPE, 40)],
            pool_hbm.at[pl.ds(pl.multiple_of(cbase + NT * COPY_STRIPE, 8),
                              40)])


@functools.cache
def _get_sc_segment_sum():
    return functools.partial(
        pl.kernel,
        mesh=plsc.VectorSubcoreMesh(core_axis_name="c", subcore_axis_name="s"),
        out_type=jax.ShapeDtypeStruct((NPAD, LATENT_DIM), jnp.float32),
        scratch_types=(
            [pltpu.VMEM((ECHUNK,), jnp.int32)] * NB
            + [pltpu.VMEM((1, ECHUNK), jnp.int32)] * NB
            + [pltpu.VMEM((ECHUNK, LATENT_DIM), jnp.float32)] * NB
            + [pltpu.VMEM((16,), jnp.int32)]
            + [pltpu.VMEM_SHARED((ACC_ROWS, LATENT_DIM), jnp.float32)]
            + [pltpu.SemaphoreType.DMA] * (2 + 2 * NB)
        ),
        compiler_params=pltpu.CompilerParams(use_tc_tiling_on_sc=False),
    )(_sc_segment_sum_body)


def _embed_body(tags_ref, w_ref, b_ref, msg_ref, cur_ref):
    tags = tags_ref[...]
    iota = lax.broadcasted_iota(jnp.int32, (BN, FEAT_DIM), 1)
    onehot = (iota == tags).astype(jnp.float32)
    msg = jnp.dot(onehot, w_ref[...],
                  preferred_element_type=jnp.float32) + b_ref[...]
    msg_ref[...] = msg
    cur_ref[...] = jnp.maximum(msg, 0.0)


def _conv_body(pool_ref, msg_ref, w_ref, b_ref, out_ref):
    x = jnp.dot(pool_ref[...], w_ref[...], preferred_element_type=jnp.float32)
    out_ref[...] = jnp.maximum(x + b_ref[...] + msg_ref[...], 0.0)


def _head_body(cur_ref, gid_ref, lab_ref, h1w_ref, h1b_ref, h2w_ref, h2b_ref,
               logits_ref, loss_ref, acc_ref, accum):
    j = pl.program_id(0)

    @pl.when(j == 0)
    def _():
        accum[...] = jnp.zeros((N_GRAPHS, LATENT_DIM), jnp.float32)

    gid = gid_ref[...]
    iota = lax.broadcasted_iota(jnp.int32, (BN, N_GRAPHS), 1)
    onehot = (iota == gid).astype(jnp.float32)
    accum[...] += lax.dot_general(onehot, cur_ref[...],
                                  (((0,), (0,)), ((), ())),
                                  preferred_element_type=jnp.float32)

    @pl.when(j == GRID_N - 1)
    def _():
        embed = jnp.maximum(accum[...], 0.0)
        h1 = jnp.maximum(
            jnp.dot(embed, h1w_ref[...], preferred_element_type=jnp.float32)
            + h1b_ref[...], 0.0)
        z = jnp.dot(h1, h2w_ref[...],
                    preferred_element_type=jnp.float32) + h2b_ref[...]
        m = jnp.max(z, axis=1, keepdims=True)
        lse = m + jnp.log(jnp.sum(jnp.exp(z - m), axis=1, keepdims=True))
        lg = z - lse
        logits_ref[...] = lg
        lab = lab_ref[...]
        pick = jnp.where(lab == 0, lg[:, 0:1], lg[:, 1:2])
        loss_ref[...] = (-jnp.mean(pick))[None, None]
        pred = (z[:, 1:2] > z[:, 0:1]).astype(jnp.int32)
        acc_ref[...] = jnp.mean((pred == lab).astype(jnp.float32))[None, None]


def _make_tc_calls(interpret=False):
    embed = pl.pallas_call(
        _embed_body,
        grid=(GRID_N,),
        in_specs=[
            pl.BlockSpec((BN, 1), lambda i: (i, 0)),
            pl.BlockSpec((FEAT_DIM, LATENT_DIM), lambda i: (0, 0)),
            pl.BlockSpec((1, LATENT_DIM), lambda i: (0, 0)),
        ],
        out_specs=[
            pl.BlockSpec((BN, LATENT_DIM), lambda i: (i, 0)),
            pl.BlockSpec((BN, LATENT_DIM), lambda i: (i, 0)),
        ],
        out_shape=[
            jax.ShapeDtypeStruct((NPAD, LATENT_DIM), jnp.float32),
            jax.ShapeDtypeStruct((NPAD, LATENT_DIM), jnp.float32),
        ],
        interpret=interpret,
    )
    conv = pl.pallas_call(
        _conv_body,
        grid=(GRID_N,),
        in_specs=[
            pl.BlockSpec((BN, LATENT_DIM), lambda i: (i, 0)),
            pl.BlockSpec((BN, LATENT_DIM), lambda i: (i, 0)),
            pl.BlockSpec((LATENT_DIM, LATENT_DIM), lambda i: (0, 0)),
            pl.BlockSpec((1, LATENT_DIM), lambda i: (0, 0)),
        ],
        out_specs=pl.BlockSpec((BN, LATENT_DIM), lambda i: (i, 0)),
        out_shape=jax.ShapeDtypeStruct((NPAD, LATENT_DIM), jnp.float32),
        interpret=interpret,
    )
    head = pl.pallas_call(
        _head_body,
        grid=(GRID_N,),
        in_specs=[
            pl.BlockSpec((BN, LATENT_DIM), lambda i: (i, 0)),
            pl.BlockSpec((BN, 1), lambda i: (i, 0)),
            pl.BlockSpec((N_GRAPHS, 1), lambda i: (0, 0)),
            pl.BlockSpec((LATENT_DIM, HIDDEN), lambda i: (0, 0)),
            pl.BlockSpec((1, HIDDEN), lambda i: (0, 0)),
            pl.BlockSpec((HIDDEN, NUM_CLASS), lambda i: (0, 0)),
            pl.BlockSpec((1, NUM_CLASS), lambda i: (0, 0)),
        ],
        out_specs=[
            pl.BlockSpec((N_GRAPHS, NUM_CLASS), lambda i: (0, 0)),
            pl.BlockSpec((1, 1), lambda i: (0, 0)),
            pl.BlockSpec((1, 1), lambda i: (0, 0)),
        ],
        out_shape=[
            jax.ShapeDtypeStruct((N_GRAPHS, NUM_CLASS), jnp.float32),
            jax.ShapeDtypeStruct((1, 1), jnp.float32),
            jax.ShapeDtypeStruct((1, 1), jnp.float32),
        ],
        scratch_shapes=[pltpu.VMEM((N_GRAPHS, LATENT_DIM), jnp.float32)],
        interpret=interpret,
    )
    return embed, conv, head


_embed_call, _conv_call, _head_call = _make_tc_calls(False)


def kernel(node_tags, edge_index, graph_ids, labels, w_n2l_w, w_n2l_b,
           conv_w, conv_b, h1_w, h1_b, h2_w, h2_b):
    tags = jnp.pad(node_tags.astype(jnp.int32),
                   (0, NPAD - N_NODES)).reshape(NPAD, 1)
    src = jnp.pad(edge_index[0].astype(jnp.int32), (0, EPAD - N_EDGES))
    dst = jnp.pad(edge_index[1].astype(jnp.int32), (0, EPAD - N_EDGES),
                  constant_values=DST_PAD)
    gids = jnp.pad(graph_ids.astype(jnp.int32), (0, NPAD - N_NODES),
                   constant_values=N_GRAPHS).reshape(NPAD, 1)
    labs = labels.astype(jnp.int32).reshape(N_GRAPHS, 1)

    srcp, dstp, cnts = _get_sc_partition()(src, dst)
    sc_segment_sum = _get_sc_segment_sum()
    msg, cur = _embed_call(tags, w_n2l_w, w_n2l_b.reshape(1, LATENT_DIM))
    for _ in range(MAX_LV):
        pool = sc_segment_sum(cur, srcp, dstp, cnts)
        cur = _conv_call(pool, msg, conv_w, conv_b.reshape(1, LATENT_DIM))
    logits, loss, acc = _head_call(
        cur, gids, labs, h1_w, h1_b.reshape(1, HIDDEN),
        h2_w, h2_b.reshape(1, NUM_CLASS))
    return logits, loss.reshape(()), acc.reshape(())


# speedup vs baseline: 1.0072x; 1.0072x over previous
import functools

import jax
import jax.numpy as jnp
from jax import lax
from jax.experimental import pallas as pl
from jax.experimental.pallas import tpu as pltpu
from jax.experimental.pallas import tpu_sc as plsc

N_NODES = 50000
N_EDGES = 800000
FEAT_DIM = 128
LATENT_DIM = 64
HIDDEN = 128
NUM_CLASS = 2
MAX_LV = 3
N_GRAPHS = 128

NPAD = 50176
BN = 512
GRID_N = NPAD // BN

ECHUNK = 128
NT = 16
NB = 3
ROWS_PER_TILE = 393
EROWS = ROWS_PER_TILE * NT
BLK = NB * ECHUNK
G_OUT = ROWS_PER_TILE // NB
EPAD = EROWS * ECHUNK

HALF = N_NODES // 2
ACC_ROWS = 25088
TRASH = 25080
DST_PAD = 1 << 20
ZSTRIPE = ACC_ROWS // NT
COPY_STRIPE = 1560


REG = 51072
LCH = 96
LNB = 4
PART_TOT = 2 * NT * REG
PBUF = 800


def _sc_partition_body(src_hbm, dst_hbm, srcp_hbm, dstp_hbm, cnt_hbm,
                       srcb0, srcb1, srcb2, dstb0, dstb1, dstb2,
                       pbuf, qbuf, cntv, srcp_s, dstp_s,
                       isemA, isemB):
    srcbs = [srcb0, srcb1, srcb2]
    dstbs = [dstb0, dstb1, dstb2]
    c = lax.axis_index("c")
    t = lax.axis_index("s")
    cbase = c * HALF
    tebase = t * (ROWS_PER_TILE * ECHUNK)

    def outer(g, carry):
        cnt, outpos = carry
        idx_hs = []
        for b in range(NB):
            base = pl.multiple_of(tebase + g * BLK + b * ECHUNK, 8)
            idx_hs.append(pltpu.async_copy(
                src_hbm.at[pl.ds(base, ECHUNK)], srcbs[b], isemA))
            idx_hs.append(pltpu.async_copy(
                dst_hbm.at[pl.ds(base, ECHUNK)], dstbs[b], isemB))
        for h in idx_hs:
            h.wait()
        ii = lax.iota(jnp.int32, 16)
        for b in range(NB):
            for i in range(8):
                s = srcbs[b][pl.ds(i * 16, 16)]
                d = dstbs[b][pl.ds(i * 16, 16)]
                rel = d - cbase
                ok = jnp.logical_and(rel >= 0, rel < HALF)
                _, sv, _ = plsc.sort_key_val(ii, s, mask=ok)
                _, rv, _ = plsc.sort_key_val(ii, rel, mask=ok)
                pbuf[pl.ds(cnt, 16)] = sv
                qbuf[pl.ds(cnt, 16)] = rv
                cnt = cnt + jnp.max(plsc.all_reduce_population_count(ok))
        do = cnt >= 384

        @pl.when(do)
        def _():
            pltpu.sync_copy(pbuf.at[pl.ds(0, 384)],
                            srcp_s.at[t, pl.ds(pl.multiple_of(outpos, 8), 384)])
            pltpu.sync_copy(qbuf.at[pl.ds(0, 384)],
                            dstp_s.at[t, pl.ds(pl.multiple_of(outpos, 8), 384)])
            for i in range(24):
                pbuf[pl.ds(i * 16, 16)] = pbuf[pl.ds(384 + i * 16, 16)]
                qbuf[pl.ds(i * 16, 16)] = qbuf[pl.ds(384 + i * 16, 16)]

        cnt = jnp.where(do, cnt - 384, cnt)
        outpos = jnp.where(do, outpos + 384, outpos)
        return (cnt, outpos)

    cnt, outpos = lax.fori_loop(
        0, G_OUT, outer, (jnp.int32(0), jnp.int32(0)))

    zero16 = jnp.zeros((16,), jnp.int32)
    trash16 = jnp.full((16,), TRASH, jnp.int32)
    for i in range(24):
        pbuf[pl.ds(cnt + i * 16, 16)] = zero16
        qbuf[pl.ds(cnt + i * 16, 16)] = trash16
    pltpu.sync_copy(pbuf.at[pl.ds(0, 384)], srcp_s.at[t, pl.ds(pl.multiple_of(outpos, 8), 384)])
    pltpu.sync_copy(qbuf.at[pl.ds(0, 384)], dstp_s.at[t, pl.ds(pl.multiple_of(outpos, 8), 384)])
    outpos = outpos + 384

    cntv[pl.ds(0, 16)] = jnp.full((16,), outpos, jnp.int32)
    pltpu.sync_copy(cntv, cnt_hbm.at[c * NT + t])
    wreg = pl.multiple_of((c * NT + t) * REG, 8)
    pltpu.sync_copy(srcp_s.at[t], srcp_hbm.at[pl.ds(wreg, REG)])
    pltpu.sync_copy(dstp_s.at[t], dstp_hbm.at[pl.ds(wreg, REG)])


@functools.cache
def _get_sc_partition():
    return functools.partial(
        pl.kernel,
        mesh=plsc.VectorSubcoreMesh(core_axis_name="c", subcore_axis_name="s"),
        out_type=[
            jax.ShapeDtypeStruct((PART_TOT,), jnp.int32),
            jax.ShapeDtypeStruct((PART_TOT,), jnp.int32),
            jax.ShapeDtypeStruct((2 * NT, 16), jnp.int32),
        ],
        scratch_types=(
            [pltpu.VMEM((ECHUNK,), jnp.int32)] * (2 * NB)
            + [pltpu.VMEM((PBUF,), jnp.int32)] * 2
            + [pltpu.VMEM((16,), jnp.int32)]
            + [pltpu.VMEM_SHARED((NT, REG), jnp.int32)] * 2
            + [pltpu.SemaphoreType.DMA] * 2
        ),
        compiler_params=pltpu.CompilerParams(use_tc_tiling_on_sc=False,
                                             needs_layout_passes=False),
    )(_sc_partition_body)


def _sc_segment_sum_body(cur_hbm, srcp_hbm, dstp_hbm, cnt_hbm, pool_hbm,
                         srcb0, srcb1, srcb2, srcb3,
                         dstbuf0, dstbuf1, dstbuf2, dstbuf3,
                         rows0, rows1, rows2, rows3,
                         cntbuf, acc,
                         isemA, isemB,
                         gsem0, gsem1, gsem2, gsem3,
                         ssem0, ssem1, ssem2, ssem3):
    srcbs = [srcb0, srcb1, srcb2, srcb3]
    rows = [rows0, rows1, rows2, rows3]
    dstbufs = [dstbuf0, dstbuf1, dstbuf2, dstbuf3]
    gsems = [gsem0, gsem1, gsem2, gsem3]
    ssems = [ssem0, ssem1, ssem2, ssem3]
    c = lax.axis_index("c")
    t = lax.axis_index("s")
    cbase = c * HALF
    rbase = (c * NT + t) * REG

    pltpu.sync_copy(cnt_hbm.at[c * NT + t], cntbuf)
    nsteps = cntbuf[pl.ds(0, 16)][0] // BLK

    zero16 = jnp.zeros((16,), jnp.float32)

    def zfill(i, carry):
        rows0[i // 4, pl.ds((i % 4) * 16, 16)] = zero16
        return carry

    lax.fori_loop(0, LCH * 4, zfill, 0)

    for k in range(16):
        pltpu.sync_copy(
            rows0,
            acc.at[pl.ds(pl.multiple_of(t * ZSTRIPE + k * LCH, 8), LCH)])
    pltpu.sync_copy(
        rows0.at[pl.ds(0, 32)],
        acc.at[pl.ds(pl.multiple_of(t * ZSTRIPE + 16 * LCH, 8), 32)])

    @pl.when(jnp.logical_and(c == 1, t == NT - 1))
    def _():
        pltpu.sync_copy(rows0, pool_hbm.at[pl.ds(N_NODES, LCH)])
        pltpu.sync_copy(rows0.at[pl.ds(0, 80)],
                        pool_hbm.at[pl.ds(N_NODES + LCH, 80)])

    plsc.subcore_barrier()

    def outer(g, carry):
        idx_hs = []
        for b in range(LNB):
            base = pl.multiple_of(rbase + g * BLK + b * LCH, 8)
            idx_hs.append(pltpu.async_copy(
                srcp_hbm.at[pl.ds(base, LCH)], srcbs[b], isemA))
            idx_hs.append(pltpu.async_copy(
                dstp_hbm.at[pl.ds(base, LCH)], dstbufs[b].at[0], isemB))
        for b in range(LNB):
            idx_hs[2 * b].wait()
            idx_hs[2 * b + 1].wait()

            @pl.when(g > 0)
            def _():
                pltpu.make_async_copy(
                    rows[b], acc.at[dstbufs[b].at[0]], ssems[b]).wait()

            pltpu.async_copy(cur_hbm.at[srcbs[b]], rows[b], gsems[b])
        for b in range(LNB):
            pltpu.make_async_copy(
                cur_hbm.at[srcbs[b]], rows[b], gsems[b]).wait()
            pltpu.async_copy(rows[b], acc.at[dstbufs[b].at[0]],
                             ssems[b], add=True)
        return carry

    lax.fori_loop(0, nsteps, outer, 0)
    for b in range(LNB):
        pltpu.make_async_copy(rows[b], acc.at[dstbufs[b].at[0]],
                              ssems[b]).wait()

    plsc.subcore_barrier()

    pltpu.sync_copy(
        acc.at[pl.ds(pl.multiple_of(t * COPY_STRIPE, 8), COPY_STRIPE)],
        pool_hbm.at[pl.ds(pl.multiple_of(cbase + t * COPY_STRIPE, 8),
                          COPY_STRIPE)])

    @pl.when(t == NT - 1)
    def _():
        pltpu.sync_copy(
            acc.at[pl.ds(NT * COPY_STRIPE, 40)],
            pool_hbm.at[pl.ds(pl.multiple_of(cbase + NT * COPY_STRIPE, 8),
                              40)])


@functools.cache
def _get_sc_segment_sum():
    return functools.partial(
        pl.kernel,
        mesh=plsc.VectorSubcoreMesh(core_axis_name="c", subcore_axis_name="s"),
        out_type=jax.ShapeDtypeStruct((NPAD, LATENT_DIM), jnp.float32),
        scratch_types=(
            [pltpu.VMEM((LCH,), jnp.int32)] * LNB
            + [pltpu.VMEM((1, LCH), jnp.int32)] * LNB
            + [pltpu.VMEM((LCH, LATENT_DIM), jnp.float32)] * LNB
            + [pltpu.VMEM((16,), jnp.int32)]
            + [pltpu.VMEM_SHARED((ACC_ROWS, LATENT_DIM), jnp.float32)]
            + [pltpu.SemaphoreType.DMA] * (2 + 2 * LNB)
        ),
        compiler_params=pltpu.CompilerParams(use_tc_tiling_on_sc=False),
    )(_sc_segment_sum_body)


def _embed_body(tags_ref, w_ref, b_ref, msg_ref, cur_ref):
    tags = tags_ref[...]
    iota = lax.broadcasted_iota(jnp.int32, (BN, FEAT_DIM), 1)
    onehot = (iota == tags).astype(jnp.float32)
    msg = jnp.dot(onehot, w_ref[...],
                  preferred_element_type=jnp.float32) + b_ref[...]
    msg_ref[...] = msg
    cur_ref[...] = jnp.maximum(msg, 0.0)


def _conv_body(pool_ref, msg_ref, w_ref, b_ref, out_ref):
    x = jnp.dot(pool_ref[...], w_ref[...], preferred_element_type=jnp.float32)
    out_ref[...] = jnp.maximum(x + b_ref[...] + msg_ref[...], 0.0)


def _head_body(cur_ref, gid_ref, lab_ref, h1w_ref, h1b_ref, h2w_ref, h2b_ref,
               logits_ref, loss_ref, acc_ref, accum):
    j = pl.program_id(0)

    @pl.when(j == 0)
    def _():
        accum[...] = jnp.zeros((N_GRAPHS, LATENT_DIM), jnp.float32)

    gid = gid_ref[...]
    iota = lax.broadcasted_iota(jnp.int32, (BN, N_GRAPHS), 1)
    onehot = (iota == gid).astype(jnp.float32)
    accum[...] += lax.dot_general(onehot, cur_ref[...],
                                  (((0,), (0,)), ((), ())),
                                  preferred_element_type=jnp.float32)

    @pl.when(j == GRID_N - 1)
    def _():
        embed = jnp.maximum(accum[...], 0.0)
        h1 = jnp.maximum(
            jnp.dot(embed, h1w_ref[...], preferred_element_type=jnp.float32)
            + h1b_ref[...], 0.0)
        z = jnp.dot(h1, h2w_ref[...],
                    preferred_element_type=jnp.float32) + h2b_ref[...]
        m = jnp.max(z, axis=1, keepdims=True)
        lse = m + jnp.log(jnp.sum(jnp.exp(z - m), axis=1, keepdims=True))
        lg = z - lse
        logits_ref[...] = lg
        lab = lab_ref[...]
        pick = jnp.where(lab == 0, lg[:, 0:1], lg[:, 1:2])
        loss_ref[...] = (-jnp.mean(pick))[None, None]
        pred = (z[:, 1:2] > z[:, 0:1]).astype(jnp.int32)
        acc_ref[...] = jnp.mean((pred == lab).astype(jnp.float32))[None, None]


def _make_tc_calls(interpret=False):
    embed = pl.pallas_call(
        _embed_body,
        grid=(GRID_N,),
        in_specs=[
            pl.BlockSpec((BN, 1), lambda i: (i, 0)),
            pl.BlockSpec((FEAT_DIM, LATENT_DIM), lambda i: (0, 0)),
            pl.BlockSpec((1, LATENT_DIM), lambda i: (0, 0)),
        ],
        out_specs=[
            pl.BlockSpec((BN, LATENT_DIM), lambda i: (i, 0)),
            pl.BlockSpec((BN, LATENT_DIM), lambda i: (i, 0)),
        ],
        out_shape=[
            jax.ShapeDtypeStruct((NPAD, LATENT_DIM), jnp.float32),
            jax.ShapeDtypeStruct((NPAD, LATENT_DIM), jnp.float32),
        ],
        interpret=interpret,
    )
    conv = pl.pallas_call(
        _conv_body,
        grid=(GRID_N,),
        in_specs=[
            pl.BlockSpec((BN, LATENT_DIM), lambda i: (i, 0)),
            pl.BlockSpec((BN, LATENT_DIM), lambda i: (i, 0)),
            pl.BlockSpec((LATENT_DIM, LATENT_DIM), lambda i: (0, 0)),
            pl.BlockSpec((1, LATENT_DIM), lambda i: (0, 0)),
        ],
        out_specs=pl.BlockSpec((BN, LATENT_DIM), lambda i: (i, 0)),
        out_shape=jax.ShapeDtypeStruct((NPAD, LATENT_DIM), jnp.float32),
        interpret=interpret,
    )
    head = pl.pallas_call(
        _head_body,
        grid=(GRID_N,),
        in_specs=[
            pl.BlockSpec((BN, LATENT_DIM), lambda i: (i, 0)),
            pl.BlockSpec((BN, 1), lambda i: (i, 0)),
            pl.BlockSpec((N_GRAPHS, 1), lambda i: (0, 0)),
            pl.BlockSpec((LATENT_DIM, HIDDEN), lambda i: (0, 0)),
            pl.BlockSpec((1, HIDDEN), lambda i: (0, 0)),
            pl.BlockSpec((HIDDEN, NUM_CLASS), lambda i: (0, 0)),
            pl.BlockSpec((1, NUM_CLASS), lambda i: (0, 0)),
        ],
        out_specs=[
            pl.BlockSpec((N_GRAPHS, NUM_CLASS), lambda i: (0, 0)),
            pl.BlockSpec((1, 1), lambda i: (0, 0)),
            pl.BlockSpec((1, 1), lambda i: (0, 0)),
        ],
        out_shape=[
            jax.ShapeDtypeStruct((N_GRAPHS, NUM_CLASS), jnp.float32),
            jax.ShapeDtypeStruct((1, 1), jnp.float32),
            jax.ShapeDtypeStruct((1, 1), jnp.float32),
        ],
        scratch_shapes=[pltpu.VMEM((N_GRAPHS, LATENT_DIM), jnp.float32)],
        interpret=interpret,
    )
    return embed, conv, head


_embed_call, _conv_call, _head_call = _make_tc_calls(False)


def kernel(node_tags, edge_index, graph_ids, labels, w_n2l_w, w_n2l_b,
           conv_w, conv_b, h1_w, h1_b, h2_w, h2_b):
    tags = jnp.pad(node_tags.astype(jnp.int32),
                   (0, NPAD - N_NODES)).reshape(NPAD, 1)
    src = jnp.pad(edge_index[0].astype(jnp.int32), (0, EPAD - N_EDGES))
    dst = jnp.pad(edge_index[1].astype(jnp.int32), (0, EPAD - N_EDGES),
                  constant_values=DST_PAD)
    gids = jnp.pad(graph_ids.astype(jnp.int32), (0, NPAD - N_NODES),
                   constant_values=N_GRAPHS).reshape(NPAD, 1)
    labs = labels.astype(jnp.int32).reshape(N_GRAPHS, 1)

    srcp, dstp, cnts = _get_sc_partition()(src, dst)
    sc_segment_sum = _get_sc_segment_sum()
    msg, cur = _embed_call(tags, w_n2l_w, w_n2l_b.reshape(1, LATENT_DIM))
    for _ in range(MAX_LV):
        pool = sc_segment_sum(cur, srcp, dstp, cnts)
        cur = _conv_call(pool, msg, conv_w, conv_b.reshape(1, LATENT_DIM))
    logits, loss, acc = _head_call(
        cur, gids, labs, h1_w, h1_b.reshape(1, HIDDEN),
        h2_w, h2_b.reshape(1, NUM_CLASS))
    return logits, loss.reshape(()), acc.reshape(())


# speedup vs baseline: 1.0542x; 1.0467x over previous
import functools

import jax
import jax.numpy as jnp
from jax import lax
from jax.experimental import pallas as pl
from jax.experimental.pallas import tpu as pltpu
from jax.experimental.pallas import tpu_sc as plsc

N_NODES = 50000
N_EDGES = 800000
FEAT_DIM = 128
LATENT_DIM = 64
HIDDEN = 128
NUM_CLASS = 2
MAX_LV = 3
N_GRAPHS = 128

NPAD = 50176
BN = 512
GRID_N = NPAD // BN

ECHUNK = 128
NT = 16
NB = 3
ROWS_PER_TILE = 393
EROWS = ROWS_PER_TILE * NT
BLK = NB * ECHUNK
G_OUT = ROWS_PER_TILE // NB
EPAD = EROWS * ECHUNK

HALF = N_NODES // 2
ACC_ROWS = 25088
TRASH = 25080
DST_PAD = 1 << 20
ZSTRIPE = ACC_ROWS // NT
COPY_STRIPE = 1560


REG = 51072
LCH = 96
LNB = 4
PART_TOT = 2 * NT * REG
PBUF = 800


def _sc_partition_body(src_hbm, dst_hbm, srcp_hbm, dstp_hbm, cnt_hbm,
                       srcb0, srcb1, srcb2, dstb0, dstb1, dstb2,
                       pbuf, qbuf, cntv, srcp_s, dstp_s,
                       isemA, isemB):
    srcbs = [srcb0, srcb1, srcb2]
    dstbs = [dstb0, dstb1, dstb2]
    c = lax.axis_index("c")
    t = lax.axis_index("s")
    cbase = c * HALF
    tebase = t * (ROWS_PER_TILE * ECHUNK)

    def outer(g, carry):
        cnt, outpos = carry
        idx_hs = []
        for b in range(NB):
            base = pl.multiple_of(tebase + g * BLK + b * ECHUNK, 8)
            idx_hs.append(pltpu.async_copy(
                src_hbm.at[pl.ds(base, ECHUNK)], srcbs[b], isemA))
            idx_hs.append(pltpu.async_copy(
                dst_hbm.at[pl.ds(base, ECHUNK)], dstbs[b], isemB))
        for h in idx_hs:
            h.wait()
        ii = lax.iota(jnp.int32, 16)
        for b in range(NB):
            for i in range(8):
                s = srcbs[b][pl.ds(i * 16, 16)]
                d = dstbs[b][pl.ds(i * 16, 16)]
                rel = d - cbase
                ok = jnp.logical_and(rel >= 0, rel < HALF)
                _, sv, _ = plsc.sort_key_val(ii, s, mask=ok)
                _, rv, _ = plsc.sort_key_val(ii, rel, mask=ok)
                pbuf[pl.ds(cnt, 16)] = sv
                qbuf[pl.ds(cnt, 16)] = rv
                cnt = cnt + jnp.max(plsc.all_reduce_population_count(ok))
        do = cnt >= 384

        @pl.when(do)
        def _():
            pltpu.sync_copy(pbuf.at[pl.ds(0, 384)],
                            srcp_s.at[t, pl.ds(pl.multiple_of(outpos, 8), 384)])
            pltpu.sync_copy(qbuf.at[pl.ds(0, 384)],
                            dstp_s.at[t, pl.ds(pl.multiple_of(outpos, 8), 384)])
            for i in range(24):
                pbuf[pl.ds(i * 16, 16)] = pbuf[pl.ds(384 + i * 16, 16)]
                qbuf[pl.ds(i * 16, 16)] = qbuf[pl.ds(384 + i * 16, 16)]

        cnt = jnp.where(do, cnt - 384, cnt)
        outpos = jnp.where(do, outpos + 384, outpos)
        return (cnt, outpos)

    cnt, outpos = lax.fori_loop(
        0, G_OUT, outer, (jnp.int32(0), jnp.int32(0)))

    zero16 = jnp.zeros((16,), jnp.int32)
    trash16 = jnp.full((16,), TRASH, jnp.int32)
    for i in range(24):
        pbuf[pl.ds(cnt + i * 16, 16)] = zero16
        qbuf[pl.ds(cnt + i * 16, 16)] = trash16
    pltpu.sync_copy(pbuf.at[pl.ds(0, 384)], srcp_s.at[t, pl.ds(pl.multiple_of(outpos, 8), 384)])
    pltpu.sync_copy(qbuf.at[pl.ds(0, 384)], dstp_s.at[t, pl.ds(pl.multiple_of(outpos, 8), 384)])
    outpos = outpos + 384

    cntv[pl.ds(0, 16)] = jnp.full((16,), outpos, jnp.int32)
    pltpu.sync_copy(cntv, cnt_hbm.at[c * NT + t])
    wreg = pl.multiple_of((c * NT + t) * REG, 8)
    pltpu.sync_copy(srcp_s.at[t], srcp_hbm.at[pl.ds(wreg, REG)])
    pltpu.sync_copy(dstp_s.at[t], dstp_hbm.at[pl.ds(wreg, REG)])


@functools.cache
def _get_sc_partition():
    return functools.partial(
        pl.kernel,
        mesh=plsc.VectorSubcoreMesh(core_axis_name="c", subcore_axis_name="s"),
        out_type=[
            jax.ShapeDtypeStruct((PART_TOT,), jnp.int32),
            jax.ShapeDtypeStruct((PART_TOT,), jnp.int32),
            jax.ShapeDtypeStruct((2 * NT, 16), jnp.int32),
        ],
        scratch_types=(
            [pltpu.VMEM((ECHUNK,), jnp.int32)] * (2 * NB)
            + [pltpu.VMEM((PBUF,), jnp.int32)] * 2
            + [pltpu.VMEM((16,), jnp.int32)]
            + [pltpu.VMEM_SHARED((NT, REG), jnp.int32)] * 2
            + [pltpu.SemaphoreType.DMA] * 2
        ),
        compiler_params=pltpu.CompilerParams(use_tc_tiling_on_sc=False,
                                             needs_layout_passes=False),
    )(_sc_partition_body)


def _sc_segment_sum_body(cur_hbm, srcp_hbm, dstp_hbm, cnt_hbm, pool_hbm,
                         srcb0, srcb1, srcb2, srcb3,
                         dstbuf0, dstbuf1, dstbuf2, dstbuf3,
                         rows0, rows1, rows2, rows3,
                         cntbuf, acc,
                         isemA, isemB,
                         gsem0, gsem1, gsem2, gsem3,
                         ssem0, ssem1, ssem2, ssem3):
    srcbs = [srcb0, srcb1, srcb2, srcb3]
    rows = [rows0, rows1, rows2, rows3]
    dstbufs = [dstbuf0, dstbuf1, dstbuf2, dstbuf3]
    gsems = [gsem0, gsem1, gsem2, gsem3]
    ssems = [ssem0, ssem1, ssem2, ssem3]
    c = lax.axis_index("c")
    t = lax.axis_index("s")
    cbase = c * HALF
    rbase = (c * NT + t) * REG

    pltpu.sync_copy(cnt_hbm.at[c * NT + t], cntbuf)
    nsteps = cntbuf[pl.ds(0, 16)][0] // BLK

    zero16 = jnp.zeros((16,), jnp.float32)

    def zfill(i, carry):
        rows0[i // 4, pl.ds((i % 4) * 16, 16)] = zero16
        return carry

    lax.fori_loop(0, LCH * 4, zfill, 0)

    for k in range(16):
        pltpu.sync_copy(
            rows0,
            acc.at[pl.ds(pl.multiple_of(t * ZSTRIPE + k * LCH, 8), LCH)])
    pltpu.sync_copy(
        rows0.at[pl.ds(0, 32)],
        acc.at[pl.ds(pl.multiple_of(t * ZSTRIPE + 16 * LCH, 8), 32)])

    @pl.when(jnp.logical_and(c == 1, t == NT - 1))
    def _():
        pltpu.sync_copy(rows0, pool_hbm.at[pl.ds(N_NODES, LCH)])
        pltpu.sync_copy(rows0.at[pl.ds(0, 80)],
                        pool_hbm.at[pl.ds(N_NODES + LCH, 80)])

    plsc.subcore_barrier()

    def outer(g, carry):
        idx_hs = []
        for b in range(LNB):
            base = pl.multiple_of(rbase + g * BLK + b * LCH, 8)
            idx_hs.append(pltpu.async_copy(
                srcp_hbm.at[pl.ds(base, LCH)], srcbs[b], isemA))
            idx_hs.append(pltpu.async_copy(
                dstp_hbm.at[pl.ds(base, LCH)], dstbufs[b].at[0], isemB))
        for b in range(LNB):
            idx_hs[2 * b].wait()
            idx_hs[2 * b + 1].wait()

            @pl.when(g > 0)
            def _():
                pltpu.make_async_copy(
                    rows[b], acc.at[dstbufs[b].at[0]], ssems[b]).wait()

            pltpu.async_copy(cur_hbm.at[srcbs[b]], rows[b], gsems[b])
        for b in range(LNB):
            pltpu.make_async_copy(
                cur_hbm.at[srcbs[b]], rows[b], gsems[b]).wait()
            pltpu.async_copy(rows[b], acc.at[dstbufs[b].at[0]],
                             ssems[b], add=True)
        return carry

    lax.fori_loop(0, nsteps, outer, 0)
    for b in range(LNB):
        pltpu.make_async_copy(rows[b], acc.at[dstbufs[b].at[0]],
                              ssems[b]).wait()

    plsc.subcore_barrier()

    pltpu.sync_copy(
        acc.at[pl.ds(pl.multiple_of(t * COPY_STRIPE, 8), COPY_STRIPE)],
        pool_hbm.at[pl.ds(pl.multiple_of(cbase + t * COPY_STRIPE, 8),
                          COPY_STRIPE)])

    @pl.when(t == NT - 1)
    def _():
        pltpu.sync_copy(
            acc.at[pl.ds(NT * COPY_STRIPE, 40)],
            pool_hbm.at[pl.ds(pl.multiple_of(cbase + NT * COPY_STRIPE, 8),
                              40)])


@functools.cache
def _get_sc_segment_sum():
    return functools.partial(
        pl.kernel,
        mesh=plsc.VectorSubcoreMesh(core_axis_name="c", subcore_axis_name="s"),
        out_type=jax.ShapeDtypeStruct((NPAD, LATENT_DIM), jnp.float32),
        scratch_types=(
            [pltpu.VMEM((LCH,), jnp.int32)] * LNB
            + [pltpu.VMEM((1, LCH), jnp.int32)] * LNB
            + [pltpu.VMEM((LCH, LATENT_DIM), jnp.float32)] * LNB
            + [pltpu.VMEM((16,), jnp.int32)]
            + [pltpu.VMEM_SHARED((ACC_ROWS, LATENT_DIM), jnp.float32)]
            + [pltpu.SemaphoreType.DMA] * (2 + 2 * LNB)
        ),
        compiler_params=pltpu.CompilerParams(use_tc_tiling_on_sc=False),
    )(_sc_segment_sum_body)


def _embed_body(tags_ref, w_ref, b_ref, msg_ref, cur_ref):
    tags = tags_ref[...]
    iota = lax.broadcasted_iota(jnp.int32, (BN, FEAT_DIM), 1)
    onehot = (iota == tags).astype(jnp.float32)
    msg = jnp.dot(onehot, w_ref[...],
                  preferred_element_type=jnp.float32) + b_ref[...]
    msg_ref[...] = msg
    cur_ref[...] = jnp.maximum(msg, 0.0)


def _conv_body(pool_ref, msg_ref, w_ref, b_ref, out_ref):
    x = jnp.dot(pool_ref[...], w_ref[...], preferred_element_type=jnp.float32)
    out_ref[...] = jnp.maximum(x + b_ref[...] + msg_ref[...], 0.0)


def _head_body(pool_ref, msg_ref, cw_ref, cb_ref, gid_ref, lab_ref,
               h1w_ref, h1b_ref, h2w_ref, h2b_ref,
               logits_ref, loss_ref, acc_ref, accum):
    j = pl.program_id(0)

    @pl.when(j == 0)
    def _():
        accum[...] = jnp.zeros((N_GRAPHS, LATENT_DIM), jnp.float32)

    cur = jnp.maximum(
        jnp.dot(pool_ref[...], cw_ref[...], preferred_element_type=jnp.float32)
        + cb_ref[...] + msg_ref[...], 0.0)
    gid = gid_ref[...]
    iota = lax.broadcasted_iota(jnp.int32, (BN, N_GRAPHS), 1)
    onehot = (iota == gid).astype(jnp.float32)
    accum[...] += lax.dot_general(onehot, cur,
                                  (((0,), (0,)), ((), ())),
                                  preferred_element_type=jnp.float32)

    @pl.when(j == GRID_N - 1)
    def _():
        embed = jnp.maximum(accum[...], 0.0)
        h1 = jnp.maximum(
            jnp.dot(embed, h1w_ref[...], preferred_element_type=jnp.float32)
            + h1b_ref[...], 0.0)
        z = jnp.dot(h1, h2w_ref[...],
                    preferred_element_type=jnp.float32) + h2b_ref[...]
        m = jnp.max(z, axis=1, keepdims=True)
        lse = m + jnp.log(jnp.sum(jnp.exp(z - m), axis=1, keepdims=True))
        lg = z - lse
        logits_ref[...] = lg
        lab = lab_ref[...]
        pick = jnp.where(lab == 0, lg[:, 0:1], lg[:, 1:2])
        loss_ref[...] = (-jnp.mean(pick))[None, None]
        pred = (z[:, 1:2] > z[:, 0:1]).astype(jnp.int32)
        acc_ref[...] = jnp.mean((pred == lab).astype(jnp.float32))[None, None]


def _make_tc_calls(interpret=False):
    embed = pl.pallas_call(
        _embed_body,
        grid=(GRID_N,),
        in_specs=[
            pl.BlockSpec((BN, 1), lambda i: (i, 0)),
            pl.BlockSpec((FEAT_DIM, LATENT_DIM), lambda i: (0, 0)),
            pl.BlockSpec((1, LATENT_DIM), lambda i: (0, 0)),
        ],
        out_specs=[
            pl.BlockSpec((BN, LATENT_DIM), lambda i: (i, 0)),
            pl.BlockSpec((BN, LATENT_DIM), lambda i: (i, 0)),
        ],
        out_shape=[
            jax.ShapeDtypeStruct((NPAD, LATENT_DIM), jnp.float32),
            jax.ShapeDtypeStruct((NPAD, LATENT_DIM), jnp.float32),
        ],
        interpret=interpret,
    )
    conv = pl.pallas_call(
        _conv_body,
        grid=(GRID_N,),
        in_specs=[
            pl.BlockSpec((BN, LATENT_DIM), lambda i: (i, 0)),
            pl.BlockSpec((BN, LATENT_DIM), lambda i: (i, 0)),
            pl.BlockSpec((LATENT_DIM, LATENT_DIM), lambda i: (0, 0)),
            pl.BlockSpec((1, LATENT_DIM), lambda i: (0, 0)),
        ],
        out_specs=pl.BlockSpec((BN, LATENT_DIM), lambda i: (i, 0)),
        out_shape=jax.ShapeDtypeStruct((NPAD, LATENT_DIM), jnp.float32),
        interpret=interpret,
    )
    head = pl.pallas_call(
        _head_body,
        grid=(GRID_N,),
        in_specs=[
            pl.BlockSpec((BN, LATENT_DIM), lambda i: (i, 0)),
            pl.BlockSpec((BN, LATENT_DIM), lambda i: (i, 0)),
            pl.BlockSpec((LATENT_DIM, LATENT_DIM), lambda i: (0, 0)),
            pl.BlockSpec((1, LATENT_DIM), lambda i: (0, 0)),
            pl.BlockSpec((BN, 1), lambda i: (i, 0)),
            pl.BlockSpec((N_GRAPHS, 1), lambda i: (0, 0)),
            pl.BlockSpec((LATENT_DIM, HIDDEN), lambda i: (0, 0)),
            pl.BlockSpec((1, HIDDEN), lambda i: (0, 0)),
            pl.BlockSpec((HIDDEN, NUM_CLASS), lambda i: (0, 0)),
            pl.BlockSpec((1, NUM_CLASS), lambda i: (0, 0)),
        ],
        out_specs=[
            pl.BlockSpec((N_GRAPHS, NUM_CLASS), lambda i: (0, 0)),
            pl.BlockSpec((1, 1), lambda i: (0, 0)),
            pl.BlockSpec((1, 1), lambda i: (0, 0)),
        ],
        out_shape=[
            jax.ShapeDtypeStruct((N_GRAPHS, NUM_CLASS), jnp.float32),
            jax.ShapeDtypeStruct((1, 1), jnp.float32),
            jax.ShapeDtypeStruct((1, 1), jnp.float32),
        ],
        scratch_shapes=[pltpu.VMEM((N_GRAPHS, LATENT_DIM), jnp.float32)],
        interpret=interpret,
    )
    return embed, conv, head


_embed_call, _conv_call, _head_call = _make_tc_calls(False)


def kernel(node_tags, edge_index, graph_ids, labels, w_n2l_w, w_n2l_b,
           conv_w, conv_b, h1_w, h1_b, h2_w, h2_b):
    tags = jnp.pad(node_tags.astype(jnp.int32),
                   (0, NPAD - N_NODES)).reshape(NPAD, 1)
    src = jnp.pad(edge_index[0].astype(jnp.int32), (0, EPAD - N_EDGES))
    dst = jnp.pad(edge_index[1].astype(jnp.int32), (0, EPAD - N_EDGES),
                  constant_values=DST_PAD)
    gids = jnp.pad(graph_ids.astype(jnp.int32), (0, NPAD - N_NODES),
                   constant_values=N_GRAPHS).reshape(NPAD, 1)
    labs = labels.astype(jnp.int32).reshape(N_GRAPHS, 1)

    srcp, dstp, cnts = _get_sc_partition()(src, dst)
    sc_segment_sum = _get_sc_segment_sum()
    msg, cur = _embed_call(tags, w_n2l_w, w_n2l_b.reshape(1, LATENT_DIM))
    cb = conv_b.reshape(1, LATENT_DIM)
    for _ in range(MAX_LV - 1):
        pool = sc_segment_sum(cur, srcp, dstp, cnts)
        cur = _conv_call(pool, msg, conv_w, cb)
    pool = sc_segment_sum(cur, srcp, dstp, cnts)
    logits, loss, acc = _head_call(
        pool, msg, conv_w, cb, gids, labs, h1_w, h1_b.reshape(1, HIDDEN),
        h2_w, h2_b.reshape(1, NUM_CLASS))
    return logits, loss.reshape(()), acc.reshape(())
